# baseline (speedup 1.0000x reference)
import jax
import jax.numpy as jnp
from jax import lax
from jax.experimental import pallas as pl
from jax.experimental.pallas import tpu as pltpu
from jax.experimental.pallas import tpu_sc as plsc

VOCAB = 1000000
EMBED_DIM = 32
BATCH = 16384
HIST = 50

NC = 2
NS = 16
NW = NC * NS

ROWS_PER_BLK = 128 // EMBED_DIM
NBLK = VOCAB // ROWS_PER_BLK

BB = BATCH // NW
CH = 256
NCHUNK = BB // CH
L = 16


def _body(x_hbm, table_hbm, out_hbm, xblk, idxh, qh, rows, cols,
          xsem, gsem, ssem):
  wid = lax.axis_index("s") * NC + lax.axis_index("c")
  b0 = wid * BB

  pltpu.make_async_copy(x_hbm.at[pl.ds(b0 * HIST, BB * HIST)], xblk,
                        xsem).start()

  lane = lax.iota(jnp.int32, L)
  lane_h = lane * HIST

  NSTEP = HIST * NCHUNK

  def build_idx(t, slot):
    h = t // NCHUNK
    c = t % NCHUNK

    @plsc.parallel_loop(0, CH // L, unroll=8)
    def jstep(j):
      v = plsc.load_gather(
          xblk, [lane_h + ((c * CH + j * L) * HIST + h)])
      idxh[slot][pl.ds(j * L, L)] = lax.shift_right_logical(v, 2)
      qh[slot][pl.ds(j * L, L)] = lax.shift_left(
          jnp.bitwise_and(v, 3), 5)

  def gather_desc(slot):
    return pltpu.make_async_copy(
        table_hbm.at[idxh[slot]], rows[slot], gsem.at[slot])

  def store_desc(t, slot):
    h = t // NCHUNK
    c = t % NCHUNK
    return pltpu.make_async_copy(
        cols[slot], out_hbm.at[h, :, pl.ds(b0 + c * CH, CH)], ssem.at[slot])

  def transpose(slot):
    for e in range(EMBED_DIM):
      ev = jnp.full((L,), e, jnp.int32)

      @plsc.parallel_loop(0, CH // L, unroll=8)
      def jstep(j, ev=ev, slot=slot):
        qv = qh[slot][pl.ds(j * L, L)]
        v = plsc.load_gather(rows[slot], [lane + j * L, qv + ev])
        cols[slot][e, pl.ds(j * L, L)] = v

  pltpu.make_async_copy(x_hbm.at[pl.ds(b0 * HIST, BB * HIST)], xblk,
                        xsem).wait()

  build_idx(0, 0)
  gather_desc(0).start()
  build_idx(1, 1)
  gather_desc(1).start()

  @pl.loop(0, NSTEP, step=2)
  def _tloop(t0):
    for s in range(2):
      t = t0 + s
      tn = t + 2
      gather_desc(s).wait()

      @pl.when(t >= 2)
      def _():
        store_desc(t - 2, s).wait()

      @pl.when(tn < NSTEP)
      def _():
        build_idx(tn, s)

      transpose(s)

      @pl.when(tn < NSTEP)
      def _():
        gather_desc(s).start()

      store_desc(t, s).start()

  store_desc(NSTEP - 2, 0).wait()
  store_desc(NSTEP - 1, 1).wait()


@jax.jit
def _embed(x_flat, table_blk):
  mesh = plsc.VectorSubcoreMesh(core_axis_name="c", subcore_axis_name="s")
  return pl.kernel(
      _body,
      out_type=jax.ShapeDtypeStruct((HIST, EMBED_DIM, BATCH), jnp.float32),
      mesh=mesh,
      scratch_types=[
          pltpu.VMEM((BB * HIST,), jnp.int32),
          [pltpu.VMEM((CH,), jnp.int32) for _ in range(2)],
          [pltpu.VMEM((CH,), jnp.int32) for _ in range(2)],
          [pltpu.VMEM((CH, 128), jnp.float32) for _ in range(2)],
          [pltpu.VMEM((EMBED_DIM, CH), jnp.float32) for _ in range(2)],
          pltpu.SemaphoreType.DMA,
          pltpu.SemaphoreType.DMA((2,)),
          pltpu.SemaphoreType.DMA((2,)),
      ],
      compiler_params=pltpu.CompilerParams(
          use_tc_tiling_on_sc=True, needs_layout_passes=False),
  )(x_flat, table_blk)


def kernel(x, table):
  x_flat = x.reshape(-1).astype(jnp.int32)
  table_blk = table.reshape(NBLK, 128)
  out = _embed(x_flat, table_blk)
  return out.transpose(2, 0, 1)


# speedup vs baseline: 1.0714x; 1.0714x over previous
import jax
import jax.numpy as jnp
from jax import lax
from jax.experimental import pallas as pl
from jax.experimental.pallas import tpu as pltpu
from jax.experimental.pallas import tpu_sc as plsc

VOCAB = 1000000
EMBED_DIM = 32
BATCH = 16384
HIST = 50

NC = 2
NS = 16
NW = NC * NS

ROWS_PER_BLK = 128 // EMBED_DIM
NBLK = VOCAB // ROWS_PER_BLK

BB = BATCH // NW
CH = 128
NCHUNK = BB // CH
L = 16


def _body(x_hbm, table_hbm, out_hbm, xblk, idxh, qh, rows, cols,
          xsem, gsem, ssem):
  wid = lax.axis_index("s") * NC + lax.axis_index("c")
  b0 = wid * BB

  pltpu.make_async_copy(x_hbm.at[pl.ds(b0 * HIST, BB * HIST)], xblk,
                        xsem).start()

  lane = lax.iota(jnp.int32, L)
  lane_h = lane * HIST

  NSTEP = HIST * NCHUNK

  def build_idx(t, slot):
    h = t // NCHUNK
    c = t % NCHUNK

    @plsc.parallel_loop(0, CH // L, unroll=8)
    def jstep(j):
      v = plsc.load_gather(
          xblk, [lane_h + ((c * CH + j * L) * HIST + h)])
      idxh[slot][pl.ds(j * L, L)] = lax.shift_right_logical(v, 2)
      qh[slot][pl.ds(j * L, L)] = lax.shift_left(
          jnp.bitwise_and(v, 3), 5)

  def gather_desc(slot):
    return pltpu.make_async_copy(
        table_hbm.at[idxh[slot]], rows[slot], gsem.at[slot])

  def store_desc(t, slot):
    h = t // NCHUNK
    c = t % NCHUNK
    return pltpu.make_async_copy(
        cols[slot], out_hbm.at[h, :, pl.ds(b0 + c * CH, CH)], ssem.at[slot])

  def transpose(slot):
    for e in range(EMBED_DIM):
      ev = jnp.full((L,), e, jnp.int32)

      @plsc.parallel_loop(0, CH // L, unroll=8)
      def jstep(j, ev=ev, slot=slot):
        qv = qh[slot][pl.ds(j * L, L)]
        v = plsc.load_gather(rows[slot], [lane + j * L, qv + ev])
        cols[slot][e, pl.ds(j * L, L)] = v

  pltpu.make_async_copy(x_hbm.at[pl.ds(b0 * HIST, BB * HIST)], xblk,
                        xsem).wait()

  build_idx(0, 0)
  gather_desc(0).start()
  build_idx(1, 1)
  gather_desc(1).start()

  @pl.loop(0, NSTEP, step=2)
  def _tloop(t0):
    for s in range(2):
      t = t0 + s
      tn = t + 2
      gather_desc(s).wait()

      @pl.when(t >= 2)
      def _():
        store_desc(t - 2, s).wait()

      transpose(s)

      @pl.when(tn < NSTEP)
      def _():
        build_idx(tn, s)
        gather_desc(s).start()

      store_desc(t, s).start()

  store_desc(NSTEP - 2, 0).wait()
  store_desc(NSTEP - 1, 1).wait()


@jax.jit
def _embed(x_flat, table_blk):
  mesh = plsc.VectorSubcoreMesh(core_axis_name="c", subcore_axis_name="s")
  return pl.kernel(
      _body,
      out_type=jax.ShapeDtypeStruct((HIST, EMBED_DIM, BATCH), jnp.float32),
      mesh=mesh,
      scratch_types=[
          pltpu.VMEM((BB * HIST,), jnp.int32),
          [pltpu.VMEM((CH,), jnp.int32) for _ in range(2)],
          [pltpu.VMEM((CH,), jnp.int32) for _ in range(2)],
          [pltpu.VMEM((CH, 128), jnp.float32) for _ in range(2)],
          [pltpu.VMEM((EMBED_DIM, CH), jnp.float32) for _ in range(2)],
          pltpu.SemaphoreType.DMA,
          pltpu.SemaphoreType.DMA((2,)),
          pltpu.SemaphoreType.DMA((2,)),
      ],
      compiler_params=pltpu.CompilerParams(
          use_tc_tiling_on_sc=True, needs_layout_passes=False),
  )(x_flat, table_blk)


def kernel(x, table):
  x_flat = x.reshape(-1).astype(jnp.int32)
  table_blk = table.reshape(NBLK, 128)
  out = _embed(x_flat, table_blk)
  return out.transpose(2, 0, 1)
